# baseline (speedup 1.0000x reference)
import functools

import jax
import jax.numpy as jnp
from jax import lax
from jax.experimental import pallas as pl
from jax.experimental.pallas import tpu as pltpu
from jax.experimental.pallas import tpu_sc as plsc

_S = 64.0
_M = 0.4

_CH = 16000
_NB = 4
_UNROLL = 8


@functools.cache
def _sc_cosface(rows, vocab):
    info = plsc.get_sparse_core_info()
    nc, lanes = info.num_cores, info.num_lanes
    nw = nc * info.num_subcores
    per_w = rows // nw
    n_per_w = per_w * vocab
    nch = n_per_w // _CH
    assert n_per_w % _CH == 0 and _CH % (lanes * _UNROLL) == 0
    assert nch % _NB == 0 and nch // _NB >= 3
    mesh = plsc.VectorSubcoreMesh(core_axis_name="c", subcore_axis_name="s")

    @functools.partial(
        pl.kernel,
        out_type=jax.ShapeDtypeStruct((rows * vocab,), jnp.float32),
        mesh=mesh,
        scratch_types=(
            [pltpu.VMEM((_CH,), jnp.float32) for _ in range(2 * _NB)]
            + [
                pltpu.VMEM((per_w,), jnp.int32),
                pltpu.VMEM((per_w,), jnp.int32),
                pltpu.VMEM((per_w,), jnp.float32),
            ]
            + [pltpu.SemaphoreType.DMA for _ in range(2 * _NB + 1)]
        ),
    )
    def cosface(x_hbm, lab_hbm, o_hbm, *scratch):
        ibufs = scratch[:_NB]
        obufs = scratch[_NB : 2 * _NB]
        lab_v, idx_v, val_v = scratch[2 * _NB : 2 * _NB + 3]
        isems = scratch[2 * _NB + 3 : 3 * _NB + 3]
        osems = scratch[3 * _NB + 3 : 4 * _NB + 3]
        fsem = scratch[4 * _NB + 3]

        wid = lax.axis_index("s") * nc + lax.axis_index("c")
        elem0 = wid * n_per_w

        def start_in(idx, b):
            pltpu.make_async_copy(
                x_hbm.at[pl.ds(elem0 + idx * _CH, _CH)], ibufs[b], isems[b]
            ).start()

        def start_out(idx, b):
            pltpu.make_async_copy(
                obufs[b], o_hbm.at[pl.ds(elem0 + idx * _CH, _CH)], osems[b]
            ).start()

        def wait_in(b):
            pltpu.make_async_copy(
                x_hbm.at[pl.ds(elem0, _CH)], ibufs[b], isems[b]
            ).wait()

        def wait_out(b):
            pltpu.make_async_copy(
                obufs[b], o_hbm.at[pl.ds(elem0, _CH)], osems[b]
            ).wait()

        def compute(b):
            ib, ob = ibufs[b], obufs[b]

            @plsc.parallel_loop(0, _CH, step=lanes, unroll=_UNROLL)
            def body(off):
                ob[pl.ds(off, lanes)] = ib[pl.ds(off, lanes)] * jnp.float32(_S)

        for b in range(_NB):
            start_in(b, b)
        for b in range(_NB):
            wait_in(b)
            compute(b)
            start_out(b, b)
            start_in(b + _NB, b)

        def group(g, carry):
            for b in range(_NB):
                idx = g * _NB + b
                wait_in(b)
                wait_out(b)
                compute(b)
                start_out(idx, b)
                start_in(idx + _NB, b)
            return carry

        lax.fori_loop(1, nch // _NB - 1, group, 0)

        for b in range(_NB):
            idx = nch - _NB + b
            wait_in(b)
            wait_out(b)
            compute(b)
            start_out(idx, b)
        for b in range(_NB):
            wait_out(b)

        row0 = wid * per_w
        pltpu.sync_copy(lab_hbm.at[pl.ds(row0, per_w)], lab_v)
        for k in range(per_w // lanes):
            lab = lab_v[pl.ds(k * lanes, lanes)]
            row = row0 + k * lanes + lax.iota(jnp.int32, lanes)
            idx_v[pl.ds(k * lanes, lanes)] = row * vocab + jnp.maximum(lab, 0)
        pltpu.async_copy(o_hbm.at[idx_v], val_v, fsem).wait()
        for k in range(per_w // lanes):
            lab = lab_v[pl.ds(k * lanes, lanes)]
            val = val_v[pl.ds(k * lanes, lanes)]
            margin = jnp.where(lab >= 0, jnp.float32(_S * _M), jnp.float32(0.0))
            val_v[pl.ds(k * lanes, lanes)] = val - margin
        pltpu.async_copy(val_v, o_hbm.at[idx_v], fsem).wait()

    return cosface


def kernel(logits, labels):
    rows, vocab = logits.shape
    out = _sc_cosface(rows, vocab)(logits.reshape(-1), labels.astype(jnp.int32))
    return out.reshape(rows, vocab)


# speedup vs baseline: 2.0557x; 2.0557x over previous
import functools

import jax
import jax.numpy as jnp
from jax import lax
from jax.experimental import pallas as pl
from jax.experimental.pallas import tpu as pltpu

_S = 64.0
_M = 0.4

_BR = 8


def _body(lab_ref, x_ref, o_ref):
    x = x_ref[...]
    lab = lab_ref[...]
    col = lax.broadcasted_iota(jnp.int32, x.shape, 1)
    margin = jnp.where(col == lab, jnp.float32(_S * _M), jnp.float32(0.0))
    o_ref[...] = x * jnp.float32(_S) - margin


@functools.cache
def _cosface(rows, vocab):
    return pl.pallas_call(
        _body,
        out_shape=jax.ShapeDtypeStruct((rows, vocab), jnp.float32),
        grid=(rows // _BR,),
        in_specs=[
            pl.BlockSpec((_BR, 1), lambda i: (i, 0)),
            pl.BlockSpec((_BR, vocab), lambda i: (i, 0)),
        ],
        out_specs=pl.BlockSpec((_BR, vocab), lambda i: (i, 0)),
        compiler_params=pltpu.CompilerParams(
            dimension_semantics=("arbitrary",),
        ),
    )


def kernel(logits, labels):
    rows, vocab = logits.shape
    lab2 = labels.astype(jnp.int32).reshape(rows, 1)
    return _cosface(rows, vocab)(lab2, logits)


# speedup vs baseline: 2.0988x; 1.0210x over previous
import functools

import jax
import jax.numpy as jnp
from jax import lax
from jax.experimental import pallas as pl
from jax.experimental.pallas import tpu as pltpu

_S = 64.0
_M = 0.4

_BR = 16


def _body(lab_ref, x_ref, o_ref):
    x = x_ref[...]
    lab = lab_ref[...]
    col = lax.broadcasted_iota(jnp.int32, x.shape, 1)
    margin = jnp.where(col == lab, jnp.float32(_S * _M), jnp.float32(0.0))
    o_ref[...] = x * jnp.float32(_S) - margin


@functools.cache
def _cosface(rows, vocab):
    return pl.pallas_call(
        _body,
        out_shape=jax.ShapeDtypeStruct((rows, vocab), jnp.float32),
        grid=(rows // _BR,),
        in_specs=[
            pl.BlockSpec((_BR, 1), lambda i: (i, 0)),
            pl.BlockSpec((_BR, vocab), lambda i: (i, 0)),
        ],
        out_specs=pl.BlockSpec((_BR, vocab), lambda i: (i, 0)),
        compiler_params=pltpu.CompilerParams(
            dimension_semantics=("arbitrary",),
        ),
    )


def kernel(logits, labels):
    rows, vocab = logits.shape
    lab2 = labels.astype(jnp.int32).reshape(rows, 1)
    return _cosface(rows, vocab)(lab2, logits)


# speedup vs baseline: 2.0995x; 1.0003x over previous
import functools

import jax
import jax.numpy as jnp
from jax import lax
from jax.experimental import pallas as pl
from jax.experimental.pallas import tpu as pltpu

_S = 64.0
_M = 0.4

_NQ = 4
_CR = 8
_NS = 2


@functools.cache
def _cosface(rows, vocab):
    rs = rows // _NQ
    nch = rs // _CR
    nsteps = nch // _NS
    assert rows % (_NQ * _CR * _NS) == 0

    def body(lab_ref, x_hbm, o_hbm, *scratch):
        n = _NQ * _NS
        ibufs = scratch[:n]
        obufs = scratch[n : 2 * n]
        isems = scratch[2 * n : 3 * n]
        osems = scratch[3 * n : 4 * n]

        def buf(bufs, q, s):
            return bufs[q * _NS + s]

        g = pl.program_id(0)

        def row0(q, s):
            return q * rs + (g * _NS + s) * _CR

        def start_in(q, s, chunk_off):
            pltpu.make_async_copy(
                x_hbm.at[pl.ds(row0(q, s) + chunk_off, _CR), :],
                buf(ibufs, q, s),
                buf(isems, q, s),
            ).start()

        def wait_in(q, s):
            pltpu.make_async_copy(
                x_hbm.at[pl.ds(0, _CR), :], buf(ibufs, q, s), buf(isems, q, s)
            ).wait()

        def start_out(q, s):
            pltpu.make_async_copy(
                buf(obufs, q, s),
                o_hbm.at[pl.ds(row0(q, s), _CR), :],
                buf(osems, q, s),
            ).start()

        def wait_out(q, s):
            pltpu.make_async_copy(
                buf(obufs, q, s), o_hbm.at[pl.ds(0, _CR), :], buf(osems, q, s)
            ).wait()

        @pl.when(g == 0)
        def _():
            for q in range(_NQ):
                for s in range(_NS):
                    start_in(q, s, 0)

        for q in range(_NQ):
            for s in range(_NS):
                wait_in(q, s)

                @pl.when(g > 0)
                def _(q=q, s=s):
                    wait_out(q, s)

                x = buf(ibufs, q, s)[...]
                lab = lab_ref[pl.ds(row0(q, s), _CR), :]
                col = lax.broadcasted_iota(jnp.int32, x.shape, 1)
                margin = jnp.where(
                    col == lab, jnp.float32(_S * _M), jnp.float32(0.0)
                )
                buf(obufs, q, s)[...] = x * jnp.float32(_S) - margin
                start_out(q, s)

                @pl.when(g < nsteps - 1)
                def _(q=q, s=s):
                    start_in(q, s, _NS * _CR)

        @pl.when(g == nsteps - 1)
        def _():
            for q in range(_NQ):
                for s in range(_NS):
                    wait_out(q, s)

    n = _NQ * _NS
    return pl.pallas_call(
        body,
        out_shape=jax.ShapeDtypeStruct((rows, vocab), jnp.float32),
        grid=(nsteps,),
        in_specs=[
            pl.BlockSpec((rows, 1), lambda g: (0, 0)),
            pl.BlockSpec(memory_space=pltpu.HBM),
        ],
        out_specs=pl.BlockSpec(memory_space=pltpu.HBM),
        scratch_shapes=(
            [pltpu.VMEM((_CR, vocab), jnp.float32) for _ in range(2 * n)]
            + [pltpu.SemaphoreType.DMA for _ in range(2 * n)]
        ),
        compiler_params=pltpu.CompilerParams(
            dimension_semantics=("arbitrary",),
        ),
    )


def kernel(logits, labels):
    rows, vocab = logits.shape
    lab2 = labels.astype(jnp.int32).reshape(rows, 1)
    return _cosface(rows, vocab)(lab2, logits)
